# baseline (speedup 1.0000x reference)
import functools

import jax
import jax.numpy as jnp
from jax import lax
from jax.experimental import pallas as pl
from jax.experimental.pallas import tpu as pltpu
from jax.experimental.pallas import tpu_sc as plsc

_B = 16384
_FEAT = 128
_NCLASS = 9
_NC = 2
_NS = 16
_NW = _NC * _NS
_RPW = _B // _NW
_NGRP = _RPW // 16


def _vsqrt(a):
    y = 0.5 * (a + 1.0)
    for _ in range(10):
        y = 0.5 * (y + a / y)
    return jnp.where(a > 0.0, y, 0.0)


def _sc_body(coord_hbm, lab_hbm, c_hbm, out_hbm, xv, labv, cv, pv, sem):
    cid = lax.axis_index("c")
    sid = lax.axis_index("s")
    wid = sid * _NC + cid
    base = wid * _RPW
    pltpu.sync_copy(lab_hbm.at[pl.ds(base, _RPW)], labv)
    pltpu.sync_copy(c_hbm, cv)
    pltpu.async_copy(coord_hbm.at[pl.ds(base, _RPW)], xv, sem).wait()

    zeros = jnp.zeros((16,), jnp.float32)
    init = (zeros,) * 18

    iota16 = lax.iota(jnp.int32, 16)

    def group(gi, carry):
        row0 = gi * 16
        lab16 = labv[pl.ds(row0, 16)]
        d2vec = zeros
        for r in range(16):
            row = row0 + r
            lab_r = lab16[r]
            acc = zeros
            for k in range(_FEAT // 16):
                xa = xv[row, pl.ds(k * 16, 16)]
                ca = cv[lab_r, pl.ds(k * 16, 16)]
                d = xa - ca
                acc = acc + d * d
            for step in (8, 4, 2, 1):
                acc = acc + jnp.take(acc, iota16 ^ step)
            d2vec = jnp.where(iota16 == r, acc, d2vec)
        dist = _vsqrt(d2vec)
        new = []
        for ci in range(_NCLASS):
            m = lab16 == ci
            new.append(carry[ci] + jnp.where(m, dist, 0.0))
        for ci in range(_NCLASS):
            m = lab16 == ci
            new.append(carry[_NCLASS + ci] + jnp.where(m, 1.0, 0.0))
        return tuple(new)

    fin = lax.fori_loop(0, _NGRP, group, init)
    for ci in range(18):
        pv[ci, :] = fin[ci]
    pltpu.sync_copy(pv, out_hbm.at[wid])


def kernel(coordinate, labels, center):
    mesh = plsc.VectorSubcoreMesh(core_axis_name="c", subcore_axis_name="s")
    k = functools.partial(
        pl.kernel,
        mesh=mesh,
        out_type=jax.ShapeDtypeStruct((_NW, 18, 16), jnp.float32),
        scratch_types=[
            pltpu.VMEM((_RPW, _FEAT), jnp.float32),
            pltpu.VMEM((_RPW,), jnp.int32),
            pltpu.VMEM((_NCLASS, _FEAT), jnp.float32),
            pltpu.VMEM((18, 16), jnp.float32),
            pltpu.SemaphoreType.DMA,
        ],
    )(_sc_body)
    parts = k(coordinate, labels, center)
    s = jnp.sum(parts[:, :_NCLASS, :], axis=(0, 2))
    cnt = jnp.sum(parts[:, _NCLASS : 2 * _NCLASS, :], axis=(0, 2))
    return jnp.sum(jnp.where(cnt > 0.0, s / cnt, 0.0)) / _B


# speedup vs baseline: 7.7378x; 7.7378x over previous
import jax
import jax.numpy as jnp
from jax.experimental import pallas as pl
from jax.experimental.pallas import tpu as pltpu

_B = 16384
_FEAT = 128
_CPAD = 16
_BLK = 8192
_GRID = _B // _BLK

_DN_T = (((1,), (1,)), ((), ()))


def _body(x_ref, lab_ref, c_ref, out_ref, acc_ref):
    i = pl.program_id(0)
    x = x_ref[...]
    c = c_ref[...]
    cpad = jnp.concatenate([c, jnp.zeros((_CPAD - 9, _FEAT), jnp.float32)], axis=0)
    lab8 = lab_ref[0]

    g_t = jax.lax.dot_general(
        cpad, x, _DN_T, preferred_element_type=jnp.float32
    )
    rown_t = jax.lax.dot_general(
        jnp.ones((1, _FEAT), jnp.float32), x * x, _DN_T,
        preferred_element_type=jnp.float32,
    )
    cn2 = jnp.sum(cpad * cpad, axis=1, keepdims=True)

    d2_t = jnp.maximum(rown_t + cn2 - 2.0 * g_t, 0.0)
    dist_t = jnp.sqrt(d2_t)

    classes = jax.lax.broadcasted_iota(jnp.int32, (_CPAD, _BLK // 8), 0)
    onehot_t = jnp.concatenate(
        [(lab8[s : s + 1, :] == classes).astype(jnp.float32) for s in range(8)],
        axis=1,
    )
    ones_row = jnp.ones((1, _BLK), jnp.float32)
    s = jax.lax.dot_general(
        onehot_t * dist_t, ones_row, _DN_T, preferred_element_type=jnp.float32
    )
    cnt = jax.lax.dot_general(
        onehot_t, ones_row, _DN_T, preferred_element_type=jnp.float32
    )

    part = jnp.concatenate([s, cnt], axis=1)
    prev = jnp.where(i == 0, jnp.zeros_like(part), acc_ref[...])
    acc = prev + part
    acc_ref[...] = acc

    @pl.when(i == pl.num_programs(0) - 1)
    def _():
        s_t = acc[:, 0:1]
        c_t = acc[:, 1:2]
        contrib = jnp.where(c_t > 0.0, s_t / c_t, 0.0)
        out_ref[0, 0] = jnp.sum(contrib) / _B


def kernel(coordinate, labels, center):
    lab3 = labels.reshape(_GRID, 8, _BLK // 8)
    out = pl.pallas_call(
        _body,
        grid=(_GRID,),
        in_specs=[
            pl.BlockSpec((_BLK, _FEAT), lambda i: (i, 0)),
            pl.BlockSpec((1, 8, _BLK // 8), lambda i: (i, 0, 0)),
            pl.BlockSpec((9, _FEAT), lambda i: (0, 0)),
        ],
        out_specs=pl.BlockSpec(memory_space=pltpu.SMEM),
        out_shape=jax.ShapeDtypeStruct((1, 1), jnp.float32),
        scratch_shapes=[pltpu.VMEM((_CPAD, 2), jnp.float32)],
    )(coordinate, lab3, center)
    return out[0, 0]


# speedup vs baseline: 9.6785x; 1.2508x over previous
import jax
import jax.numpy as jnp
from jax.experimental import pallas as pl
from jax.experimental.pallas import tpu as pltpu

_B = 16384
_FEAT = 128
_CPAD = 16
_BLK = 8192
_GRID = _B // _BLK

_DN_T = (((1,), (1,)), ((), ()))


def _body(x_ref, lab_ref, c_ref, out_ref, acc_ref):
    i = pl.program_id(0)
    x = x_ref[...]
    c = c_ref[...]
    cpad = jnp.concatenate([c, jnp.zeros((_CPAD - 9, _FEAT), jnp.float32)], axis=0)
    lab = lab_ref[0]

    g_t = jax.lax.dot_general(
        cpad, x, _DN_T, preferred_element_type=jnp.float32
    )
    rown_t = jax.lax.dot_general(
        jnp.ones((1, _FEAT), jnp.float32), x * x, _DN_T,
        preferred_element_type=jnp.float32,
    )
    cn2 = jnp.sum(cpad * cpad, axis=1, keepdims=True)

    d2_t = jnp.maximum(rown_t + cn2 - 2.0 * g_t, 0.0)
    dist_t = jnp.sqrt(d2_t)

    classes = jax.lax.broadcasted_iota(jnp.int32, (_CPAD, _BLK), 0)
    onehot_t = (lab == classes).astype(jnp.float32)
    ones_row = jnp.ones((1, _BLK), jnp.float32)
    s = jax.lax.dot_general(
        onehot_t * dist_t, ones_row, _DN_T, preferred_element_type=jnp.float32
    )
    cnt = jax.lax.dot_general(
        onehot_t, ones_row, _DN_T, preferred_element_type=jnp.float32
    )

    part = jnp.concatenate([s, cnt], axis=1)
    prev = jnp.where(i == 0, jnp.zeros_like(part), acc_ref[...])
    acc = prev + part
    acc_ref[...] = acc

    @pl.when(i == pl.num_programs(0) - 1)
    def _():
        s_t = acc[:, 0:1]
        c_t = acc[:, 1:2]
        contrib = jnp.where(c_t > 0.0, s_t / c_t, 0.0)
        out_ref[0, 0] = jnp.sum(contrib) / _B


def kernel(coordinate, labels, center):
    lab3 = labels.reshape(_GRID, 1, _BLK)
    out = pl.pallas_call(
        _body,
        grid=(_GRID,),
        in_specs=[
            pl.BlockSpec((_BLK, _FEAT), lambda i: (i, 0)),
            pl.BlockSpec((1, 1, _BLK), lambda i: (i, 0, 0)),
            pl.BlockSpec((9, _FEAT), lambda i: (0, 0)),
        ],
        out_specs=pl.BlockSpec(memory_space=pltpu.SMEM),
        out_shape=jax.ShapeDtypeStruct((1, 1), jnp.float32),
        scratch_shapes=[pltpu.VMEM((_CPAD, 2), jnp.float32)],
    )(coordinate, lab3, center)
    return out[0, 0]
